# baseline (speedup 1.0000x reference)
import jax
import jax.numpy as jnp
from jax import lax
from jax.experimental import pallas as pl
from jax.experimental.pallas import tpu as pltpu
from jax.experimental.pallas import tpu_sc as plsc

N = 10000
E = 320000
D = 128
G = 16

NC = 2
NS = 16
NW = NC * NS

BN = 400


_MESH = dict(core_axis_name="c", subcore_axis_name="s",
             num_cores=NC, num_subcores=NS)


def _sc_degree(col):
    e_per = E // NW
    n_vec = e_per // 16

    def body(col_hbm, out_hbm, colbuf, hist):
        c = lax.axis_index("c")
        s = lax.axis_index("s")
        wid = c * NS + s
        z = jnp.zeros((16,), jnp.float32)
        ones = jnp.ones((16,), jnp.float32)

        def zero_body(i, carry):
            hist[pl.ds(pl.multiple_of(i * 16, 16), 16)] = z
            return carry

        lax.fori_loop(0, N // 16, zero_body, 0)
        pltpu.sync_copy(col_hbm.at[pl.ds(wid * e_per, e_per)], colbuf)

        def acc_body(i, carry):
            idx = colbuf[pl.ds(pl.multiple_of(i * 16, 16), 16)]
            plsc.addupdate_scatter(hist, [idx], ones)
            return carry

        lax.fori_loop(0, n_vec, acc_body, 0)
        pltpu.sync_copy(hist, out_hbm.at[wid])

    return pl.kernel(
        body,
        out_type=jax.ShapeDtypeStruct((NW, N), jnp.float32),
        mesh=plsc.VectorSubcoreMesh(**_MESH),
        compiler_params=pltpu.CompilerParams(needs_layout_passes=False),
        scratch_types=[
            pltpu.VMEM((e_per,), jnp.int32),
            pltpu.VMEM((N,), jnp.float32),
        ],
    )(col)


_CH = 96
_NBUF = 4
_DC = 80
_NRC = N // _DC
_KMAX = -(-_NRC // NS)
_EPT = E // NW
_NCH = -(-_EPT // _CH)
_NA = N + 8


def _sc_scatter(hp, edges4):
    n_ch = _NCH
    n_grp = n_ch // _NBUF
    tail = n_ch - n_grp * _NBUF

    def body(hp_hbm, edges_hbm, out_hbm, *refs):
        ibufs = list(refs[:_NBUF])
        rbufs = list(refs[_NBUF:2 * _NBUF])
        acc = refs[2 * _NBUF]
        isems = list(refs[2 * _NBUF + 1:2 * _NBUF + 1 + _NBUF])
        gsems = list(refs[2 * _NBUF + 1 + _NBUF:2 * _NBUF + 1 + 2 * _NBUF])
        ssems = list(refs[2 * _NBUF + 1 + 2 * _NBUF:])
        c = lax.axis_index("c")
        s = lax.axis_index("s")
        wid = c * NS + s
        r0 = rbufs[0]
        z = jnp.zeros((16,), jnp.float32)

        def iload(j, b):
            pltpu.async_copy(edges_hbm.at[wid, j], ibufs[b], isems[b])

        def iload_wait(j, b):
            pltpu.make_async_copy(edges_hbm.at[wid, j], ibufs[b],
                                  isems[b]).wait()

        def gather(j, b):
            iload_wait(j, b)
            pltpu.async_copy(hp_hbm.at[ibufs[b].at[0]], rbufs[b], gsems[b])

        def gather_wait(j, b):
            pltpu.make_async_copy(hp_hbm.at[ibufs[b].at[0]], rbufs[b],
                                  gsems[b]).wait()

        def scat(j, b):
            pltpu.async_copy(rbufs[b], acc.at[ibufs[b].at[1]], ssems[b],
                             add=True)

        def scat_wait(j, b):
            pltpu.make_async_copy(rbufs[b], acc.at[ibufs[b].at[1]],
                                  ssems[b]).wait()

        for b in range(_NBUF):
            iload(b, b)

        def zfill(i, carry):
            for j in range(D // 16):
                r0[i, pl.ds(j * 16, 16)] = z
            return carry

        lax.fori_loop(0, _CH, zfill, 0)
        for k in range(_KMAX):
            m = s + k * NS

            @pl.when(m < _NRC)
            def _():
                pltpu.sync_copy(r0.at[pl.ds(0, _DC)],
                                acc.at[pl.ds(m * _DC, _DC)])

        for b in range(_NBUF):
            gather(b, b)
        plsc.subcore_barrier()

        def ring(i, carry):
            for b in range(_NBUF):
                j = i * _NBUF + b
                gather_wait(j, b)
                scat(j, b)
            for b in range(_NBUF):
                j = i * _NBUF + b
                nxt = j + _NBUF

                @pl.when(nxt < n_ch)
                def _():
                    scat_wait(j, b)
                    iload(nxt, b)
                    gather(nxt, b)
            return carry

        lax.fori_loop(0, n_grp, ring, 0)
        for t in range(tail):
            j = n_grp * _NBUF + t
            gather_wait(j, t)
            scat(j, t)
        for b in range(_NBUF):
            scat_wait(0, b)
        plsc.subcore_barrier()

        for k in range(_KMAX):
            m = s + k * NS

            @pl.when(m < _NRC)
            def _():
                st = pl.multiple_of(m * _DC, 8)
                pltpu.sync_copy(acc.at[pl.ds(st, _DC)], r0.at[pl.ds(0, _DC)])
                pltpu.sync_copy(r0.at[pl.ds(0, _DC)],
                                out_hbm.at[c, pl.ds(st, _DC)])

    sems = [pltpu.SemaphoreType.DMA] * (3 * _NBUF)
    return pl.kernel(
        body,
        out_type=jax.ShapeDtypeStruct((NC, N, D), jnp.float32),
        mesh=plsc.VectorSubcoreMesh(**_MESH),
        scratch_types=(
            [pltpu.VMEM((2, _CH), jnp.int32)] * _NBUF
            + [pltpu.VMEM((_CH, D), jnp.float32)] * _NBUF
            + [pltpu.VMEM_SHARED((_NA, D), jnp.float32)]
            + sems
        ),
    )(hp, edges4)


def _tc_first_body(x_ref, w_ref, degp_ref, hp_ref, dinv_ref):
    deg = jnp.sum(degp_ref[...], axis=1, keepdims=True) + 1.0
    dinv = lax.rsqrt(deg)
    h = jnp.dot(x_ref[...], w_ref[...], preferred_element_type=jnp.float32)
    hp_ref[...] = h * dinv
    dinv_ref[...] = dinv


def _tc_first(x, w, degp_t):
    return pl.pallas_call(
        _tc_first_body,
        grid=(N // BN,),
        in_specs=[
            pl.BlockSpec((BN, D), lambda i: (i, 0)),
            pl.BlockSpec((D, D), lambda i: (0, 0)),
            pl.BlockSpec((BN, NW), lambda i: (i, 0)),
        ],
        out_specs=[
            pl.BlockSpec((BN, D), lambda i: (i, 0)),
            pl.BlockSpec((BN, 1), lambda i: (i, 0)),
        ],
        out_shape=[
            jax.ShapeDtypeStruct((N, D), jnp.float32),
            jax.ShapeDtypeStruct((N, 1), jnp.float32),
        ],
    )(x, w, degp_t)


def _tc_mid_body(s_ref, hp_ref, dinv_ref, b_ref, w_ref, out_ref):
    t = s_ref[0] + s_ref[1] + hp_ref[...]
    x = jnp.maximum(t * dinv_ref[...] + b_ref[...], 0.0)
    out_ref[...] = (
        jnp.dot(x, w_ref[...], preferred_element_type=jnp.float32) * dinv_ref[...]
    )


def _tc_mid(s_parts, hp, dinv, b, w):
    return pl.pallas_call(
        _tc_mid_body,
        grid=(N // BN,),
        in_specs=[
            pl.BlockSpec((NC, BN, D), lambda i: (0, i, 0)),
            pl.BlockSpec((BN, D), lambda i: (i, 0)),
            pl.BlockSpec((BN, 1), lambda i: (i, 0)),
            pl.BlockSpec((1, D), lambda i: (0, 0)),
            pl.BlockSpec((D, D), lambda i: (0, 0)),
        ],
        out_specs=pl.BlockSpec((BN, D), lambda i: (i, 0)),
        out_shape=jax.ShapeDtypeStruct((N, D), jnp.float32),
    )(s_parts, hp, dinv, b, w)


def _tc_head_body(s_ref, hp_ref, dinv_ref, b3_ref, wf1_ref, bf1_ref, batch_ref,
                  pool_ref, cnt_ref):
    i = pl.program_id(0)
    t = s_ref[0] + s_ref[1] + hp_ref[...]
    x3 = jnp.maximum(t * dinv_ref[...] + b3_ref[...], 0.0)
    h = jnp.dot(x3, wf1_ref[...], preferred_element_type=jnp.float32)
    h = jnp.maximum(h + bf1_ref[...], 0.0)
    onehot = (batch_ref[...] == lax.broadcasted_iota(jnp.int32, (BN, G), 1))
    onehot = onehot.astype(jnp.float32)
    psum = lax.dot_general(onehot, h, (((0,), (0,)), ((), ())),
                           preferred_element_type=jnp.float32)

    @pl.when(i == 0)
    def _():
        pool_ref[...] = jnp.zeros_like(pool_ref)
        cnt_ref[...] = jnp.zeros_like(cnt_ref)

    pool_ref[...] += psum
    cnt_ref[...] += jnp.sum(onehot, axis=0, keepdims=True)


def _tc_head(s_parts, hp, dinv, b3, wf1, bf1, batch2d):
    return pl.pallas_call(
        _tc_head_body,
        grid=(N // BN,),
        in_specs=[
            pl.BlockSpec((NC, BN, D), lambda i: (0, i, 0)),
            pl.BlockSpec((BN, D), lambda i: (i, 0)),
            pl.BlockSpec((BN, 1), lambda i: (i, 0)),
            pl.BlockSpec((1, D), lambda i: (0, 0)),
            pl.BlockSpec((D, D), lambda i: (0, 0)),
            pl.BlockSpec((1, D), lambda i: (0, 0)),
            pl.BlockSpec((BN, 1), lambda i: (i, 0)),
        ],
        out_specs=[
            pl.BlockSpec((G, D), lambda i: (0, 0)),
            pl.BlockSpec((1, G), lambda i: (0, 0)),
        ],
        out_shape=[
            jax.ShapeDtypeStruct((G, D), jnp.float32),
            jax.ShapeDtypeStruct((1, G), jnp.float32),
        ],
    )(s_parts, hp, dinv, b3, wf1, bf1, batch2d)


def _tc_final_body(pool_ref, cnt_ref, wf2_ref, bf2_ref, out_ref):
    cnt = jnp.transpose(cnt_ref[...])
    mean = pool_ref[...] / jnp.maximum(cnt, 1.0)
    logits = jnp.dot(mean, wf2_ref[...], preferred_element_type=jnp.float32)
    logits = logits + bf2_ref[...]
    m = jnp.max(logits, axis=1, keepdims=True)
    ex = jnp.exp(logits - m)
    out_ref[...] = ex / jnp.sum(ex, axis=1, keepdims=True)


def _tc_final(pool, cnt, wf2, bf2):
    return pl.pallas_call(
        _tc_final_body,
        out_shape=jax.ShapeDtypeStruct((G, 2), jnp.float32),
    )(pool, cnt, wf2, bf2)


def kernel(x, edge_index, batch, W1, b1, W2, b2, W3, b3, Wf1, bf1, Wf2, bf2):
    row = edge_index[0]
    col = edge_index[1]
    pad = _NCH * _CH - _EPT
    rowp = jnp.pad(row.reshape(NW, _EPT), ((0, 0), (0, pad)))
    colp = jnp.pad(col.reshape(NW, _EPT), ((0, 0), (0, pad)),
                   constant_values=N)
    edges4 = jnp.stack([rowp.reshape(NW, _NCH, _CH),
                        colp.reshape(NW, _NCH, _CH)], axis=2)

    degp = _sc_degree(col)
    degp_t = degp.T

    hp1, dinv = _tc_first(x, W1, degp_t)
    s1 = _sc_scatter(hp1, edges4)
    hp2 = _tc_mid(s1, hp1, dinv, b1.reshape(1, D), W2)
    s2 = _sc_scatter(hp2, edges4)
    hp3 = _tc_mid(s2, hp2, dinv, b2.reshape(1, D), W3)
    s3 = _sc_scatter(hp3, edges4)
    pool, cnt = _tc_head(s3, hp3, dinv, b3.reshape(1, D), Wf1,
                         bf1.reshape(1, D), batch.reshape(N, 1))
    return _tc_final(pool, cnt, Wf2, bf2.reshape(1, 2))


# speedup vs baseline: 1.4815x; 1.4815x over previous
import jax
import jax.numpy as jnp
from jax import lax
from jax.experimental import pallas as pl
from jax.experimental.pallas import tpu as pltpu
from jax.experimental.pallas import tpu_sc as plsc

N = 10000
E = 320000
D = 128
G = 16

NC = 2
NS = 16
NW = NC * NS

BN = 400


_MESH = dict(core_axis_name="c", subcore_axis_name="s",
             num_cores=NC, num_subcores=NS)


def _sc_degree(col):
    e_per = E // NW
    n_vec = e_per // 16

    def body(col_hbm, out_hbm, colbuf, hist):
        c = lax.axis_index("c")
        s = lax.axis_index("s")
        wid = c * NS + s
        z = jnp.zeros((16,), jnp.float32)
        ones = jnp.ones((16,), jnp.float32)

        def zero_body(i, carry):
            hist[pl.ds(pl.multiple_of(i * 16, 16), 16)] = z
            return carry

        lax.fori_loop(0, N // 16, zero_body, 0)
        pltpu.sync_copy(col_hbm.at[pl.ds(wid * e_per, e_per)], colbuf)

        def acc_body(i, carry):
            idx = colbuf[pl.ds(pl.multiple_of(i * 16, 16), 16)]
            plsc.addupdate_scatter(hist, [idx], ones)
            return carry

        lax.fori_loop(0, n_vec, acc_body, 0)
        pltpu.sync_copy(hist, out_hbm.at[wid])

    return pl.kernel(
        body,
        out_type=jax.ShapeDtypeStruct((NW, N), jnp.float32),
        mesh=plsc.VectorSubcoreMesh(**_MESH),
        compiler_params=pltpu.CompilerParams(needs_layout_passes=False),
        scratch_types=[
            pltpu.VMEM((e_per,), jnp.int32),
            pltpu.VMEM((N,), jnp.float32),
        ],
    )(col)


_CH = 80
_NBUF = 4
_DC = 80
_NRC = N // _DC
_KMAX = -(-_NRC // NS)
_EPT = E // NW
_NCH = -(-_EPT // _CH)
_NA = N + 8


def _sc_scatter(hp, edges4):
    n_ch = _NCH
    n_grp = n_ch // _NBUF
    tail = n_ch - n_grp * _NBUF

    def body(hp_hbm, edges_hbm, out_hbm, *refs):
        ibufs = list(refs[:_NBUF])
        rbufs = list(refs[_NBUF:2 * _NBUF])
        acc = refs[2 * _NBUF]
        isems = list(refs[2 * _NBUF + 1:2 * _NBUF + 1 + _NBUF])
        gsems = list(refs[2 * _NBUF + 1 + _NBUF:2 * _NBUF + 1 + 2 * _NBUF])
        ssems = list(refs[2 * _NBUF + 1 + 2 * _NBUF:])
        c = lax.axis_index("c")
        s = lax.axis_index("s")
        wid = c * NS + s
        r0 = rbufs[0]
        z = jnp.zeros((16,), jnp.float32)

        def iload(j, b):
            pltpu.async_copy(edges_hbm.at[wid, j], ibufs[b], isems[b])

        def iload_wait(j, b):
            pltpu.make_async_copy(edges_hbm.at[wid, j], ibufs[b],
                                  isems[b]).wait()

        def gather(j, b):
            iload_wait(j, b)
            pltpu.async_copy(hp_hbm.at[ibufs[b].at[0]], rbufs[b], gsems[b])

        def gather_wait(j, b):
            pltpu.make_async_copy(hp_hbm.at[ibufs[b].at[0]], rbufs[b],
                                  gsems[b]).wait()

        def scat(j, b):
            pltpu.async_copy(rbufs[b], acc.at[ibufs[b].at[1]], ssems[b],
                             add=True)

        def scat_wait(j, b):
            pltpu.make_async_copy(rbufs[b], acc.at[ibufs[b].at[1]],
                                  ssems[b]).wait()

        for b in range(_NBUF):
            iload(b, b)

        def zfill(i, carry):
            for j in range(D // 16):
                r0[i, pl.ds(j * 16, 16)] = z
            return carry

        lax.fori_loop(0, _CH, zfill, 0)
        for k in range(_KMAX):
            m = s + k * NS

            @pl.when(m < _NRC)
            def _():
                pltpu.sync_copy(r0.at[pl.ds(0, _DC)],
                                acc.at[pl.ds(m * _DC, _DC)])

        for b in range(_NBUF):
            gather(b, b)
        plsc.subcore_barrier()

        def ring(i, carry):
            for b in range(_NBUF):
                j = i * _NBUF + b
                gather_wait(j, b)
                scat(j, b)
            for b in range(_NBUF):
                j = i * _NBUF + b
                nxt = j + _NBUF

                @pl.when(nxt < n_ch)
                def _():
                    scat_wait(j, b)
                    iload(nxt, b)
                    gather(nxt, b)
            return carry

        lax.fori_loop(0, n_grp, ring, 0)
        for t in range(tail):
            j = n_grp * _NBUF + t
            gather_wait(j, t)
            scat(j, t)
        for b in range(_NBUF):
            scat_wait(0, b)
        plsc.subcore_barrier()

        for k in range(_KMAX):
            m = s + k * NS

            @pl.when(m < _NRC)
            def _():
                st = pl.multiple_of(m * _DC, 8)
                pltpu.sync_copy(acc.at[pl.ds(st, _DC)], r0.at[pl.ds(0, _DC)])
                pltpu.sync_copy(r0.at[pl.ds(0, _DC)],
                                out_hbm.at[c, pl.ds(st, _DC)])

    sems = [pltpu.SemaphoreType.DMA] * (3 * _NBUF)
    return pl.kernel(
        body,
        out_type=jax.ShapeDtypeStruct((NC, N, D), jnp.float32),
        mesh=plsc.VectorSubcoreMesh(**_MESH),
        scratch_types=(
            [pltpu.VMEM((2, _CH), jnp.int32)] * _NBUF
            + [pltpu.VMEM((_CH, D), jnp.float32)] * _NBUF
            + [pltpu.VMEM_SHARED((_NA, D), jnp.float32)]
            + sems
        ),
    )(hp, edges4)


def _tc_first_body(x_ref, w_ref, degp_ref, hp_ref, dinv_ref):
    deg = jnp.sum(degp_ref[...], axis=1, keepdims=True) + 1.0
    dinv = lax.rsqrt(deg)
    h = jnp.dot(x_ref[...], w_ref[...], preferred_element_type=jnp.float32)
    hp_ref[...] = h * dinv
    dinv_ref[...] = dinv


def _tc_first(x, w, degp_t):
    return pl.pallas_call(
        _tc_first_body,
        grid=(N // BN,),
        in_specs=[
            pl.BlockSpec((BN, D), lambda i: (i, 0)),
            pl.BlockSpec((D, D), lambda i: (0, 0)),
            pl.BlockSpec((BN, NW), lambda i: (i, 0)),
        ],
        out_specs=[
            pl.BlockSpec((BN, D), lambda i: (i, 0)),
            pl.BlockSpec((BN, 1), lambda i: (i, 0)),
        ],
        out_shape=[
            jax.ShapeDtypeStruct((N, D), jnp.float32),
            jax.ShapeDtypeStruct((N, 1), jnp.float32),
        ],
    )(x, w, degp_t)


def _tc_mid_body(s_ref, hp_ref, dinv_ref, b_ref, w_ref, out_ref):
    t = s_ref[0] + s_ref[1] + hp_ref[...]
    x = jnp.maximum(t * dinv_ref[...] + b_ref[...], 0.0)
    out_ref[...] = (
        jnp.dot(x, w_ref[...], preferred_element_type=jnp.float32) * dinv_ref[...]
    )


def _tc_mid(s_parts, hp, dinv, b, w):
    return pl.pallas_call(
        _tc_mid_body,
        grid=(N // BN,),
        in_specs=[
            pl.BlockSpec((NC, BN, D), lambda i: (0, i, 0)),
            pl.BlockSpec((BN, D), lambda i: (i, 0)),
            pl.BlockSpec((BN, 1), lambda i: (i, 0)),
            pl.BlockSpec((1, D), lambda i: (0, 0)),
            pl.BlockSpec((D, D), lambda i: (0, 0)),
        ],
        out_specs=pl.BlockSpec((BN, D), lambda i: (i, 0)),
        out_shape=jax.ShapeDtypeStruct((N, D), jnp.float32),
    )(s_parts, hp, dinv, b, w)


def _tc_head_body(s_ref, hp_ref, dinv_ref, b3_ref, wf1_ref, bf1_ref, batch_ref,
                  pool_ref, cnt_ref):
    i = pl.program_id(0)
    t = s_ref[0] + s_ref[1] + hp_ref[...]
    x3 = jnp.maximum(t * dinv_ref[...] + b3_ref[...], 0.0)
    h = jnp.dot(x3, wf1_ref[...], preferred_element_type=jnp.float32)
    h = jnp.maximum(h + bf1_ref[...], 0.0)
    onehot = (batch_ref[...] == lax.broadcasted_iota(jnp.int32, (BN, G), 1))
    onehot = onehot.astype(jnp.float32)
    psum = lax.dot_general(onehot, h, (((0,), (0,)), ((), ())),
                           preferred_element_type=jnp.float32)

    @pl.when(i == 0)
    def _():
        pool_ref[...] = jnp.zeros_like(pool_ref)
        cnt_ref[...] = jnp.zeros_like(cnt_ref)

    pool_ref[...] += psum
    cnt_ref[...] += jnp.sum(onehot, axis=0, keepdims=True)


def _tc_head(s_parts, hp, dinv, b3, wf1, bf1, batch2d):
    return pl.pallas_call(
        _tc_head_body,
        grid=(N // BN,),
        in_specs=[
            pl.BlockSpec((NC, BN, D), lambda i: (0, i, 0)),
            pl.BlockSpec((BN, D), lambda i: (i, 0)),
            pl.BlockSpec((BN, 1), lambda i: (i, 0)),
            pl.BlockSpec((1, D), lambda i: (0, 0)),
            pl.BlockSpec((D, D), lambda i: (0, 0)),
            pl.BlockSpec((1, D), lambda i: (0, 0)),
            pl.BlockSpec((BN, 1), lambda i: (i, 0)),
        ],
        out_specs=[
            pl.BlockSpec((G, D), lambda i: (0, 0)),
            pl.BlockSpec((1, G), lambda i: (0, 0)),
        ],
        out_shape=[
            jax.ShapeDtypeStruct((G, D), jnp.float32),
            jax.ShapeDtypeStruct((1, G), jnp.float32),
        ],
    )(s_parts, hp, dinv, b3, wf1, bf1, batch2d)


def _tc_final_body(pool_ref, cnt_ref, wf2_ref, bf2_ref, out_ref):
    cnt = jnp.transpose(cnt_ref[...])
    mean = pool_ref[...] / jnp.maximum(cnt, 1.0)
    logits = jnp.dot(mean, wf2_ref[...], preferred_element_type=jnp.float32)
    logits = logits + bf2_ref[...]
    m = jnp.max(logits, axis=1, keepdims=True)
    ex = jnp.exp(logits - m)
    out_ref[...] = ex / jnp.sum(ex, axis=1, keepdims=True)


def _tc_final(pool, cnt, wf2, bf2):
    return pl.pallas_call(
        _tc_final_body,
        out_shape=jax.ShapeDtypeStruct((G, 2), jnp.float32),
    )(pool, cnt, wf2, bf2)


def kernel(x, edge_index, batch, W1, b1, W2, b2, W3, b3, Wf1, bf1, Wf2, bf2):
    row = edge_index[0]
    col = edge_index[1]
    pad = _NCH * _CH - _EPT
    rowp = jnp.pad(row.reshape(NW, _EPT), ((0, 0), (0, pad)))
    colp = jnp.pad(col.reshape(NW, _EPT), ((0, 0), (0, pad)),
                   constant_values=N)
    edges4 = jnp.stack([rowp.reshape(NW, _NCH, _CH),
                        colp.reshape(NW, _NCH, _CH)], axis=2)

    degp = _sc_degree(col)
    degp_t = degp.T

    hp1, dinv = _tc_first(x, W1, degp_t)
    s1 = _sc_scatter(hp1, edges4)
    hp2 = _tc_mid(s1, hp1, dinv, b1.reshape(1, D), W2)
    s2 = _sc_scatter(hp2, edges4)
    hp3 = _tc_mid(s2, hp2, dinv, b2.reshape(1, D), W3)
    s3 = _sc_scatter(hp3, edges4)
    pool, cnt = _tc_head(s3, hp3, dinv, b3.reshape(1, D), Wf1,
                         bf1.reshape(1, D), batch.reshape(N, 1))
    return _tc_final(pool, cnt, Wf2, bf2.reshape(1, 2))


# speedup vs baseline: 1.4912x; 1.0065x over previous
import jax
import jax.numpy as jnp
from jax import lax
from jax.experimental import pallas as pl
from jax.experimental.pallas import tpu as pltpu
from jax.experimental.pallas import tpu_sc as plsc

N = 10000
E = 320000
D = 128
G = 16

NC = 2
NS = 16
NW = NC * NS

BN = 400


_MESH = dict(core_axis_name="c", subcore_axis_name="s",
             num_cores=NC, num_subcores=NS)


def _sc_degree(col):
    e_per = E // NW
    n_vec = e_per // 16

    def body(col_hbm, out_hbm, colbuf, hist):
        c = lax.axis_index("c")
        s = lax.axis_index("s")
        wid = c * NS + s
        z = jnp.zeros((16,), jnp.float32)
        ones = jnp.ones((16,), jnp.float32)

        def zero_body(i, carry):
            hist[pl.ds(pl.multiple_of(i * 16, 16), 16)] = z
            return carry

        lax.fori_loop(0, N // 16, zero_body, 0)
        pltpu.sync_copy(col_hbm.at[pl.ds(wid * e_per, e_per)], colbuf)

        def acc_body(i, carry):
            idx = colbuf[pl.ds(pl.multiple_of(i * 16, 16), 16)]
            plsc.addupdate_scatter(hist, [idx], ones)
            return carry

        lax.fori_loop(0, n_vec, acc_body, 0)
        pltpu.sync_copy(hist, out_hbm.at[wid])

    return pl.kernel(
        body,
        out_type=jax.ShapeDtypeStruct((NW, N), jnp.float32),
        mesh=plsc.VectorSubcoreMesh(**_MESH),
        compiler_params=pltpu.CompilerParams(needs_layout_passes=False),
        scratch_types=[
            pltpu.VMEM((e_per,), jnp.int32),
            pltpu.VMEM((N,), jnp.float32),
        ],
    )(col)


_CH = 80
_NBUF = 4
_DC = 80
_NRC = N // _DC
_KMAX = -(-_NRC // NS)
_EPT = E // NW
_NCH = -(-_EPT // _CH)
_NA = N + 8


def _sc_scatter(hp, edges4):
    n_ch = _NCH
    n_grp = n_ch // _NBUF
    tail = n_ch - n_grp * _NBUF

    def body(hp_hbm, edges_hbm, out_hbm, *refs):
        ibufs = list(refs[:_NBUF])
        rbufs = list(refs[_NBUF:2 * _NBUF])
        acc = refs[2 * _NBUF]
        isems = list(refs[2 * _NBUF + 1:2 * _NBUF + 1 + _NBUF])
        gsems = list(refs[2 * _NBUF + 1 + _NBUF:2 * _NBUF + 1 + 2 * _NBUF])
        ssems = list(refs[2 * _NBUF + 1 + 2 * _NBUF:])
        c = lax.axis_index("c")
        s = lax.axis_index("s")
        wid = c * NS + s
        r0 = rbufs[0]
        z = jnp.zeros((16,), jnp.float32)

        def iload(j, b):
            pltpu.async_copy(edges_hbm.at[wid, j], ibufs[b], isems[b])

        def iload_wait(j, b):
            pltpu.make_async_copy(edges_hbm.at[wid, j], ibufs[b],
                                  isems[b]).wait()

        def gather(j, b):
            iload_wait(j, b)
            pltpu.async_copy(hp_hbm.at[ibufs[b].at[0]], rbufs[b], gsems[b])

        def gather_wait(j, b):
            pltpu.make_async_copy(hp_hbm.at[ibufs[b].at[0]], rbufs[b],
                                  gsems[b]).wait()

        def scat(j, b):
            pltpu.async_copy(rbufs[b], acc.at[ibufs[b].at[1]], ssems[b],
                             add=True)

        def scat_wait(j, b):
            pltpu.make_async_copy(rbufs[b], acc.at[ibufs[b].at[1]],
                                  ssems[b]).wait()

        for b in range(_NBUF):
            iload(b, b)

        def zfill(i, carry):
            for j in range(D // 16):
                r0[i, pl.ds(j * 16, 16)] = z
            return carry

        lax.fori_loop(0, _CH, zfill, 0)
        zsem = isems[0]
        for k in range(_KMAX):
            m = s + k * NS

            @pl.when(m < _NRC)
            def _():
                pltpu.async_copy(r0.at[pl.ds(0, _DC)],
                                 acc.at[pl.ds(m * _DC, _DC)], zsem)
        for k in range(_KMAX):
            m = s + k * NS

            @pl.when(m < _NRC)
            def _():
                pltpu.make_async_copy(r0.at[pl.ds(0, _DC)],
                                      acc.at[pl.ds(m * _DC, _DC)],
                                      zsem).wait()

        for b in range(_NBUF):
            gather(b, b)
        plsc.subcore_barrier()

        def ring(i, carry):
            for b in range(_NBUF):
                j = i * _NBUF + b
                gather_wait(j, b)
                scat(j, b)
            for b in range(_NBUF):
                j = i * _NBUF + b
                nxt = j + _NBUF

                @pl.when(nxt < n_ch)
                def _():
                    scat_wait(j, b)
                    iload(nxt, b)
                    gather(nxt, b)
            return carry

        lax.fori_loop(0, n_grp, ring, 0)
        for t in range(tail):
            j = n_grp * _NBUF + t
            gather_wait(j, t)
            scat(j, t)
        for b in range(_NBUF):
            scat_wait(0, b)
        plsc.subcore_barrier()

        dsem = isems[0]
        for k in range(_KMAX):
            m = s + k * NS

            @pl.when(m < _NRC)
            def _():
                st = pl.multiple_of(m * _DC, 8)
                pltpu.async_copy(acc.at[pl.ds(st, _DC)],
                                 out_hbm.at[c, pl.ds(st, _DC)], dsem)
        for k in range(_KMAX):
            m = s + k * NS

            @pl.when(m < _NRC)
            def _():
                st = pl.multiple_of(m * _DC, 8)
                pltpu.make_async_copy(acc.at[pl.ds(st, _DC)],
                                      out_hbm.at[c, pl.ds(st, _DC)],
                                      dsem).wait()

    sems = [pltpu.SemaphoreType.DMA] * (3 * _NBUF)
    return pl.kernel(
        body,
        out_type=jax.ShapeDtypeStruct((NC, N, D), jnp.float32),
        mesh=plsc.VectorSubcoreMesh(**_MESH),
        scratch_types=(
            [pltpu.VMEM((2, _CH), jnp.int32)] * _NBUF
            + [pltpu.VMEM((_CH, D), jnp.float32)] * _NBUF
            + [pltpu.VMEM_SHARED((_NA, D), jnp.float32)]
            + sems
        ),
    )(hp, edges4)


def _tc_first_body(x_ref, w_ref, degp_ref, hp_ref, dinv_ref):
    deg = jnp.sum(degp_ref[...], axis=1, keepdims=True) + 1.0
    dinv = lax.rsqrt(deg)
    h = jnp.dot(x_ref[...], w_ref[...], preferred_element_type=jnp.float32)
    hp_ref[...] = h * dinv
    dinv_ref[...] = dinv


def _tc_first(x, w, degp_t):
    return pl.pallas_call(
        _tc_first_body,
        grid=(N // BN,),
        in_specs=[
            pl.BlockSpec((BN, D), lambda i: (i, 0)),
            pl.BlockSpec((D, D), lambda i: (0, 0)),
            pl.BlockSpec((BN, NW), lambda i: (i, 0)),
        ],
        out_specs=[
            pl.BlockSpec((BN, D), lambda i: (i, 0)),
            pl.BlockSpec((BN, 1), lambda i: (i, 0)),
        ],
        out_shape=[
            jax.ShapeDtypeStruct((N, D), jnp.float32),
            jax.ShapeDtypeStruct((N, 1), jnp.float32),
        ],
    )(x, w, degp_t)


def _tc_mid_body(s_ref, hp_ref, dinv_ref, b_ref, w_ref, out_ref):
    t = s_ref[0] + s_ref[1] + hp_ref[...]
    x = jnp.maximum(t * dinv_ref[...] + b_ref[...], 0.0)
    out_ref[...] = (
        jnp.dot(x, w_ref[...], preferred_element_type=jnp.float32) * dinv_ref[...]
    )


def _tc_mid(s_parts, hp, dinv, b, w):
    return pl.pallas_call(
        _tc_mid_body,
        grid=(N // BN,),
        in_specs=[
            pl.BlockSpec((NC, BN, D), lambda i: (0, i, 0)),
            pl.BlockSpec((BN, D), lambda i: (i, 0)),
            pl.BlockSpec((BN, 1), lambda i: (i, 0)),
            pl.BlockSpec((1, D), lambda i: (0, 0)),
            pl.BlockSpec((D, D), lambda i: (0, 0)),
        ],
        out_specs=pl.BlockSpec((BN, D), lambda i: (i, 0)),
        out_shape=jax.ShapeDtypeStruct((N, D), jnp.float32),
    )(s_parts, hp, dinv, b, w)


def _tc_head_body(s_ref, hp_ref, dinv_ref, b3_ref, wf1_ref, bf1_ref, batch_ref,
                  pool_ref, cnt_ref):
    i = pl.program_id(0)
    t = s_ref[0] + s_ref[1] + hp_ref[...]
    x3 = jnp.maximum(t * dinv_ref[...] + b3_ref[...], 0.0)
    h = jnp.dot(x3, wf1_ref[...], preferred_element_type=jnp.float32)
    h = jnp.maximum(h + bf1_ref[...], 0.0)
    onehot = (batch_ref[...] == lax.broadcasted_iota(jnp.int32, (BN, G), 1))
    onehot = onehot.astype(jnp.float32)
    psum = lax.dot_general(onehot, h, (((0,), (0,)), ((), ())),
                           preferred_element_type=jnp.float32)

    @pl.when(i == 0)
    def _():
        pool_ref[...] = jnp.zeros_like(pool_ref)
        cnt_ref[...] = jnp.zeros_like(cnt_ref)

    pool_ref[...] += psum
    cnt_ref[...] += jnp.sum(onehot, axis=0, keepdims=True)


def _tc_head(s_parts, hp, dinv, b3, wf1, bf1, batch2d):
    return pl.pallas_call(
        _tc_head_body,
        grid=(N // BN,),
        in_specs=[
            pl.BlockSpec((NC, BN, D), lambda i: (0, i, 0)),
            pl.BlockSpec((BN, D), lambda i: (i, 0)),
            pl.BlockSpec((BN, 1), lambda i: (i, 0)),
            pl.BlockSpec((1, D), lambda i: (0, 0)),
            pl.BlockSpec((D, D), lambda i: (0, 0)),
            pl.BlockSpec((1, D), lambda i: (0, 0)),
            pl.BlockSpec((BN, 1), lambda i: (i, 0)),
        ],
        out_specs=[
            pl.BlockSpec((G, D), lambda i: (0, 0)),
            pl.BlockSpec((1, G), lambda i: (0, 0)),
        ],
        out_shape=[
            jax.ShapeDtypeStruct((G, D), jnp.float32),
            jax.ShapeDtypeStruct((1, G), jnp.float32),
        ],
    )(s_parts, hp, dinv, b3, wf1, bf1, batch2d)


def _tc_final_body(pool_ref, cnt_ref, wf2_ref, bf2_ref, out_ref):
    cnt = jnp.transpose(cnt_ref[...])
    mean = pool_ref[...] / jnp.maximum(cnt, 1.0)
    logits = jnp.dot(mean, wf2_ref[...], preferred_element_type=jnp.float32)
    logits = logits + bf2_ref[...]
    m = jnp.max(logits, axis=1, keepdims=True)
    ex = jnp.exp(logits - m)
    out_ref[...] = ex / jnp.sum(ex, axis=1, keepdims=True)


def _tc_final(pool, cnt, wf2, bf2):
    return pl.pallas_call(
        _tc_final_body,
        out_shape=jax.ShapeDtypeStruct((G, 2), jnp.float32),
    )(pool, cnt, wf2, bf2)


def kernel(x, edge_index, batch, W1, b1, W2, b2, W3, b3, Wf1, bf1, Wf2, bf2):
    row = edge_index[0]
    col = edge_index[1]
    pad = _NCH * _CH - _EPT
    rowp = jnp.pad(row.reshape(NW, _EPT), ((0, 0), (0, pad)))
    colp = jnp.pad(col.reshape(NW, _EPT), ((0, 0), (0, pad)),
                   constant_values=N)
    edges4 = jnp.stack([rowp.reshape(NW, _NCH, _CH),
                        colp.reshape(NW, _NCH, _CH)], axis=2)

    degp = _sc_degree(col)
    degp_t = degp.T

    hp1, dinv = _tc_first(x, W1, degp_t)
    s1 = _sc_scatter(hp1, edges4)
    hp2 = _tc_mid(s1, hp1, dinv, b1.reshape(1, D), W2)
    s2 = _sc_scatter(hp2, edges4)
    hp3 = _tc_mid(s2, hp2, dinv, b2.reshape(1, D), W3)
    s3 = _sc_scatter(hp3, edges4)
    pool, cnt = _tc_head(s3, hp3, dinv, b3.reshape(1, D), Wf1,
                         bf1.reshape(1, D), batch.reshape(N, 1))
    return _tc_final(pool, cnt, Wf2, bf2.reshape(1, 2))


# speedup vs baseline: 1.4933x; 1.0015x over previous
import jax
import jax.numpy as jnp
from jax import lax
from jax.experimental import pallas as pl
from jax.experimental.pallas import tpu as pltpu
from jax.experimental.pallas import tpu_sc as plsc

N = 10000
E = 320000
D = 128
G = 16

NC = 2
NS = 16
NW = NC * NS

BN = 400


_MESH = dict(core_axis_name="c", subcore_axis_name="s",
             num_cores=NC, num_subcores=NS)


def _sc_degree(col):
    e_per = E // NW
    n_vec = e_per // 16

    def body(col_hbm, out_hbm, colbuf, hist):
        c = lax.axis_index("c")
        s = lax.axis_index("s")
        wid = c * NS + s
        z = jnp.zeros((16,), jnp.float32)
        ones = jnp.ones((16,), jnp.float32)

        def zero_body(i, carry):
            hist[pl.ds(pl.multiple_of(i * 16, 16), 16)] = z
            return carry

        lax.fori_loop(0, N // 16, zero_body, 0)
        pltpu.sync_copy(col_hbm.at[pl.ds(wid * e_per, e_per)], colbuf)

        def acc_body(i, carry):
            idx = colbuf[pl.ds(pl.multiple_of(i * 16, 16), 16)]
            plsc.addupdate_scatter(hist, [idx], ones)
            return carry

        lax.fori_loop(0, n_vec, acc_body, 0)
        pltpu.sync_copy(hist, out_hbm.at[wid])

    return pl.kernel(
        body,
        out_type=jax.ShapeDtypeStruct((NW, N), jnp.float32),
        mesh=plsc.VectorSubcoreMesh(**_MESH),
        compiler_params=pltpu.CompilerParams(needs_layout_passes=False),
        scratch_types=[
            pltpu.VMEM((e_per,), jnp.int32),
            pltpu.VMEM((N,), jnp.float32),
        ],
    )(col)


_CH = 80
_NBUF = 4
_DC = 80
_NRC = N // _DC
_KMAX = -(-_NRC // NS)
_EPT = E // NW
_NCH = -(-_EPT // _CH)
_NA = N + 8


def _sc_scatter(hp, edges4):
    n_ch = _NCH
    n_grp = n_ch // _NBUF
    tail = n_ch - n_grp * _NBUF

    def body(hp_hbm, edges_hbm, out_hbm, *refs):
        ibufs = list(refs[:_NBUF])
        rbufs = list(refs[_NBUF:2 * _NBUF])
        acc = refs[2 * _NBUF]
        isems = list(refs[2 * _NBUF + 1:2 * _NBUF + 1 + _NBUF])
        gsems = list(refs[2 * _NBUF + 1 + _NBUF:2 * _NBUF + 1 + 2 * _NBUF])
        ssems = list(refs[2 * _NBUF + 1 + 2 * _NBUF:])
        c = lax.axis_index("c")
        s = lax.axis_index("s")
        wid = c * NS + s
        r0 = rbufs[0]
        z = jnp.zeros((16,), jnp.float32)

        def iload(j, b):
            pltpu.async_copy(edges_hbm.at[wid, j], ibufs[b], isems[b])

        def iload_wait(j, b):
            pltpu.make_async_copy(edges_hbm.at[wid, j], ibufs[b],
                                  isems[b]).wait()

        def gather(j, b):
            iload_wait(j, b)
            pltpu.async_copy(hp_hbm.at[ibufs[b].at[0]], rbufs[b], gsems[b])

        def gather_wait(j, b):
            pltpu.make_async_copy(hp_hbm.at[ibufs[b].at[0]], rbufs[b],
                                  gsems[b]).wait()

        def scat(j, b):
            pltpu.async_copy(rbufs[b], acc.at[ibufs[b].at[1]], ssems[b],
                             add=True)

        def scat_wait(j, b):
            pltpu.make_async_copy(rbufs[b], acc.at[ibufs[b].at[1]],
                                  ssems[b]).wait()

        for b in range(_NBUF):
            iload(b, b)

        def zfill(i, carry):
            for j in range(D // 16):
                r0[i, pl.ds(j * 16, 16)] = z
            return carry

        lax.fori_loop(0, _CH, zfill, 0)
        zsem = isems[0]
        for k in range(_KMAX):
            m = s + k * NS

            @pl.when(m < _NRC)
            def _():
                pltpu.async_copy(r0.at[pl.ds(0, _DC)],
                                 acc.at[pl.ds(m * _DC, _DC)], zsem)
        for k in range(_KMAX):
            m = s + k * NS

            @pl.when(m < _NRC)
            def _():
                pltpu.make_async_copy(r0.at[pl.ds(0, _DC)],
                                      acc.at[pl.ds(m * _DC, _DC)],
                                      zsem).wait()

        for b in range(_NBUF):
            gather(b, b)
        plsc.subcore_barrier()

        def ring(i, carry):
            for b in range(_NBUF):
                j = i * _NBUF + b
                gather_wait(j, b)
                scat(j, b)
            for b in range(_NBUF):
                j = i * _NBUF + b
                nxt = j + _NBUF

                @pl.when(nxt < n_ch)
                def _():
                    scat_wait(j, b)
                    iload(nxt, b)
                    gather(nxt, b)
            return carry

        lax.fori_loop(0, n_grp, ring, 0)
        for t in range(tail):
            j = n_grp * _NBUF + t
            gather_wait(j, t)
            scat(j, t)
        for b in range(_NBUF):
            scat_wait(0, b)
        plsc.subcore_barrier()

        dsem = isems[0]
        for k in range(_KMAX):
            m = s + k * NS

            @pl.when(m < _NRC)
            def _():
                st = pl.multiple_of(m * _DC, 8)
                pltpu.async_copy(acc.at[pl.ds(st, _DC)],
                                 out_hbm.at[c, pl.ds(st, _DC)], dsem)
        for k in range(_KMAX):
            m = s + k * NS

            @pl.when(m < _NRC)
            def _():
                st = pl.multiple_of(m * _DC, 8)
                pltpu.make_async_copy(acc.at[pl.ds(st, _DC)],
                                      out_hbm.at[c, pl.ds(st, _DC)],
                                      dsem).wait()

    sems = [pltpu.SemaphoreType.DMA] * (3 * _NBUF)
    return pl.kernel(
        body,
        out_type=jax.ShapeDtypeStruct((NC, N, D), jnp.float32),
        mesh=plsc.VectorSubcoreMesh(**_MESH),
        scratch_types=(
            [pltpu.VMEM((2, _CH), jnp.int32)] * _NBUF
            + [pltpu.VMEM((_CH, D), jnp.float32)] * _NBUF
            + [pltpu.VMEM_SHARED((_NA, D), jnp.float32)]
            + sems
        ),
    )(hp, edges4)


def _tc_first_body(x_ref, w_ref, degp_ref, hp_ref, dinv_ref):
    deg = jnp.sum(degp_ref[...], axis=1, keepdims=True) + 1.0
    dinv = lax.rsqrt(deg)
    h = jnp.dot(x_ref[...], w_ref[...], preferred_element_type=jnp.float32)
    hp_ref[...] = h * dinv
    dinv_ref[...] = dinv


def _tc_first(x, w, degp_t):
    return pl.pallas_call(
        _tc_first_body,
        grid=(N // BN,),
        in_specs=[
            pl.BlockSpec((BN, D), lambda i: (i, 0)),
            pl.BlockSpec((D, D), lambda i: (0, 0)),
            pl.BlockSpec((BN, NW), lambda i: (i, 0)),
        ],
        out_specs=[
            pl.BlockSpec((BN, D), lambda i: (i, 0)),
            pl.BlockSpec((BN, 1), lambda i: (i, 0)),
        ],
        out_shape=[
            jax.ShapeDtypeStruct((N, D), jnp.float32),
            jax.ShapeDtypeStruct((N, 1), jnp.float32),
        ],
    )(x, w, degp_t)


def _tc_mid_body(s_ref, hp_ref, dinv_ref, b_ref, w_ref, out_ref):
    t = s_ref[0] + s_ref[1] + hp_ref[...]
    x = jnp.maximum(t * dinv_ref[...] + b_ref[...], 0.0)
    out_ref[...] = (
        jnp.dot(x, w_ref[...], preferred_element_type=jnp.float32) * dinv_ref[...]
    )


def _tc_mid(s_parts, hp, dinv, b, w):
    return pl.pallas_call(
        _tc_mid_body,
        grid=(N // BN,),
        in_specs=[
            pl.BlockSpec((NC, BN, D), lambda i: (0, i, 0)),
            pl.BlockSpec((BN, D), lambda i: (i, 0)),
            pl.BlockSpec((BN, 1), lambda i: (i, 0)),
            pl.BlockSpec((1, D), lambda i: (0, 0)),
            pl.BlockSpec((D, D), lambda i: (0, 0)),
        ],
        out_specs=pl.BlockSpec((BN, D), lambda i: (i, 0)),
        out_shape=jax.ShapeDtypeStruct((N, D), jnp.float32),
    )(s_parts, hp, dinv, b, w)


def _tc_head_body(s_ref, hp_ref, dinv_ref, b3_ref, wf1_ref, bf1_ref, batch_ref,
                  wf2_ref, bf2_ref, out_ref, pool_ref, cnt_ref):
    i = pl.program_id(0)
    t = s_ref[0] + s_ref[1] + hp_ref[...]
    x3 = jnp.maximum(t * dinv_ref[...] + b3_ref[...], 0.0)
    h = jnp.dot(x3, wf1_ref[...], preferred_element_type=jnp.float32)
    h = jnp.maximum(h + bf1_ref[...], 0.0)
    onehot = (batch_ref[...] == lax.broadcasted_iota(jnp.int32, (BN, G), 1))
    onehot = onehot.astype(jnp.float32)
    psum = lax.dot_general(onehot, h, (((0,), (0,)), ((), ())),
                           preferred_element_type=jnp.float32)

    @pl.when(i == 0)
    def _():
        pool_ref[...] = jnp.zeros_like(pool_ref)
        cnt_ref[...] = jnp.zeros_like(cnt_ref)

    pool_ref[...] += psum
    cnt_ref[...] += jnp.sum(onehot, axis=0, keepdims=True)

    @pl.when(i == N // BN - 1)
    def _():
        cnt = jnp.transpose(cnt_ref[...])
        mean = pool_ref[...] / jnp.maximum(cnt, 1.0)
        logits = jnp.dot(mean, wf2_ref[...],
                         preferred_element_type=jnp.float32) + bf2_ref[...]
        m = jnp.max(logits, axis=1, keepdims=True)
        ex = jnp.exp(logits - m)
        out_ref[...] = ex / jnp.sum(ex, axis=1, keepdims=True)


def _tc_head(s_parts, hp, dinv, b3, wf1, bf1, batch2d, wf2, bf2):
    return pl.pallas_call(
        _tc_head_body,
        grid=(N // BN,),
        in_specs=[
            pl.BlockSpec((NC, BN, D), lambda i: (0, i, 0)),
            pl.BlockSpec((BN, D), lambda i: (i, 0)),
            pl.BlockSpec((BN, 1), lambda i: (i, 0)),
            pl.BlockSpec((1, D), lambda i: (0, 0)),
            pl.BlockSpec((D, D), lambda i: (0, 0)),
            pl.BlockSpec((1, D), lambda i: (0, 0)),
            pl.BlockSpec((BN, 1), lambda i: (i, 0)),
            pl.BlockSpec((D, 2), lambda i: (0, 0)),
            pl.BlockSpec((1, 2), lambda i: (0, 0)),
        ],
        out_specs=pl.BlockSpec((G, 2), lambda i: (0, 0)),
        out_shape=jax.ShapeDtypeStruct((G, 2), jnp.float32),
        scratch_shapes=[
            pltpu.VMEM((G, D), jnp.float32),
            pltpu.VMEM((1, G), jnp.float32),
        ],
    )(s_parts, hp, dinv, b3, wf1, bf1, batch2d, wf2, bf2)


def kernel(x, edge_index, batch, W1, b1, W2, b2, W3, b3, Wf1, bf1, Wf2, bf2):
    row = edge_index[0]
    col = edge_index[1]
    pad = _NCH * _CH - _EPT
    rowp = jnp.pad(row.reshape(NW, _EPT), ((0, 0), (0, pad)))
    colp = jnp.pad(col.reshape(NW, _EPT), ((0, 0), (0, pad)),
                   constant_values=N)
    edges4 = jnp.stack([rowp.reshape(NW, _NCH, _CH),
                        colp.reshape(NW, _NCH, _CH)], axis=2)

    degp = _sc_degree(col)
    degp_t = degp.T

    hp1, dinv = _tc_first(x, W1, degp_t)
    s1 = _sc_scatter(hp1, edges4)
    hp2 = _tc_mid(s1, hp1, dinv, b1.reshape(1, D), W2)
    s2 = _sc_scatter(hp2, edges4)
    hp3 = _tc_mid(s2, hp2, dinv, b2.reshape(1, D), W3)
    s3 = _sc_scatter(hp3, edges4)
    return _tc_head(s3, hp3, dinv, b3.reshape(1, D), Wf1,
                    bf1.reshape(1, D), batch.reshape(N, 1),
                    Wf2, bf2.reshape(1, 2))


# speedup vs baseline: 1.6029x; 1.0734x over previous
import jax
import jax.numpy as jnp
from jax import lax
from jax.experimental import pallas as pl
from jax.experimental.pallas import tpu as pltpu
from jax.experimental.pallas import tpu_sc as plsc

N = 10000
E = 320000
D = 128
G = 16

NC = 2
NS = 16
NW = NC * NS

BN = 400


_MESH = dict(core_axis_name="c", subcore_axis_name="s",
             num_cores=NC, num_subcores=NS)


def _sc_degree(col):
    e_per = E // NW
    n_vec = e_per // 16

    def body(col_hbm, out_hbm, colbuf, hist):
        c = lax.axis_index("c")
        s = lax.axis_index("s")
        wid = c * NS + s
        z = jnp.zeros((16,), jnp.float32)
        ones = jnp.ones((16,), jnp.float32)

        def zero_body(i, carry):
            hist[pl.ds(pl.multiple_of(i * 16, 16), 16)] = z
            return carry

        lax.fori_loop(0, N // 16, zero_body, 0)
        pltpu.sync_copy(col_hbm.at[pl.ds(wid * e_per, e_per)], colbuf)

        def acc_body(i, carry):
            idx = colbuf[pl.ds(pl.multiple_of(i * 16, 16), 16)]
            plsc.addupdate_scatter(hist, [idx], ones)
            return carry

        lax.fori_loop(0, n_vec, acc_body, 0)
        pltpu.sync_copy(hist, out_hbm.at[wid])

    return pl.kernel(
        body,
        out_type=jax.ShapeDtypeStruct((NW, N), jnp.float32),
        mesh=plsc.VectorSubcoreMesh(**_MESH),
        compiler_params=pltpu.CompilerParams(needs_layout_passes=False),
        scratch_types=[
            pltpu.VMEM((e_per,), jnp.int32),
            pltpu.VMEM((N,), jnp.float32),
        ],
    )(col)


_CH = 80
_NBUF = 4
_IB = 2 * _NBUF
_DC = 80
_NRC = N // _DC
_KMAX = -(-_NRC // NS)
_EPT = E // NW
_NCH = -(-_EPT // _CH)
_NA = N + 8


def _sc_scatter(hp, edges4):
    n_ch = _NCH
    n_grp = n_ch // _NBUF
    tail = n_ch - n_grp * _NBUF

    def body(hp_hbm, edges_hbm, out_hbm, *refs):
        ibufs = list(refs[:_IB])
        rbufs = list(refs[_IB:_IB + _NBUF])
        acc = refs[_IB + _NBUF]
        zsem = refs[_IB + _NBUF + 1]
        isems = list(refs[_IB + _NBUF + 2:_IB + _NBUF + 2 + _IB])
        gsems = list(refs[_IB + _NBUF + 2 + _IB:_IB + _NBUF + 2 + _IB + _NBUF])
        ssems = list(refs[_IB + _NBUF + 2 + _IB + _NBUF:])
        c = lax.axis_index("c")
        s = lax.axis_index("s")
        wid = c * NS + s
        r0 = rbufs[0]
        z = jnp.zeros((16,), jnp.float32)

        def iload(j, ib):
            pltpu.async_copy(edges_hbm.at[wid, j], ibufs[ib], isems[ib])

        def iload_wait(j, ib):
            pltpu.make_async_copy(edges_hbm.at[wid, j], ibufs[ib],
                                  isems[ib]).wait()

        def gather(j, b, ib):
            iload_wait(j, ib)
            pltpu.async_copy(hp_hbm.at[ibufs[ib].at[0]], rbufs[b], gsems[b])

        def gather_wait(j, b, ib):
            pltpu.make_async_copy(hp_hbm.at[ibufs[ib].at[0]], rbufs[b],
                                  gsems[b]).wait()

        def scat(j, b, ib):
            pltpu.async_copy(rbufs[b], acc.at[ibufs[ib].at[1]], ssems[b],
                             add=True)

        def scat_wait(j, b, ib):
            pltpu.make_async_copy(rbufs[b], acc.at[ibufs[ib].at[1]],
                                  ssems[b]).wait()

        for j0 in range(min(_IB, _NCH)):
            iload(j0, j0)

        def zfill(i, carry):
            for j in range(D // 16):
                r0[i, pl.ds(j * 16, 16)] = z
            return carry

        lax.fori_loop(0, _CH, zfill, 0)
        for k in range(_KMAX):
            m = s + k * NS

            @pl.when(m < _NRC)
            def _():
                pltpu.async_copy(r0.at[pl.ds(0, _DC)],
                                 acc.at[pl.ds(m * _DC, _DC)], zsem)
        for k in range(_KMAX):
            m = s + k * NS

            @pl.when(m < _NRC)
            def _():
                pltpu.make_async_copy(r0.at[pl.ds(0, _DC)],
                                      acc.at[pl.ds(m * _DC, _DC)],
                                      zsem).wait()

        for b in range(_NBUF):
            gather(b, b, b % _IB)
        plsc.subcore_barrier()

        def do_round(i, r):
            for b in range(_NBUF):
                j = i * _NBUF + b
                ib = (b + _NBUF * r) % _IB
                gather_wait(j, b, 0)
                scat(j, b, ib)
            for b in range(_NBUF):
                j = i * _NBUF + b
                ib = (b + _NBUF * r) % _IB
                nxt = j + _NBUF

                @pl.when(nxt < n_ch)
                def _():
                    scat_wait(j, b, 0)
                    nld = j + _IB

                    @pl.when(nld < n_ch)
                    def _():
                        iload(nld, ib)

                    gather(nxt, b, (b + _NBUF * (1 - r)) % _IB)

        def ring(i2, carry):
            do_round(2 * i2, 0)
            do_round(2 * i2 + 1, 1)
            return carry

        lax.fori_loop(0, n_grp // 2, ring, 0)
        for i in range(2 * (n_grp // 2), n_grp):
            do_round(i, i % 2)
        for t in range(tail):
            j = n_grp * _NBUF + t
            gather_wait(j, t, 0)
            scat(j, t, j % _IB)
        for b in range(_NBUF):
            scat_wait(0, b, 0)
        plsc.subcore_barrier()

        dsem = zsem
        for k in range(_KMAX):
            m = s + k * NS

            @pl.when(m < _NRC)
            def _():
                st = pl.multiple_of(m * _DC, 8)
                pltpu.async_copy(acc.at[pl.ds(st, _DC)],
                                 out_hbm.at[c, pl.ds(st, _DC)], dsem)
        for k in range(_KMAX):
            m = s + k * NS

            @pl.when(m < _NRC)
            def _():
                st = pl.multiple_of(m * _DC, 8)
                pltpu.make_async_copy(acc.at[pl.ds(st, _DC)],
                                      out_hbm.at[c, pl.ds(st, _DC)],
                                      dsem).wait()

    sems = [pltpu.SemaphoreType.DMA] * (_IB + 2 * _NBUF)
    return pl.kernel(
        body,
        out_type=jax.ShapeDtypeStruct((NC, N, D), jnp.float32),
        mesh=plsc.VectorSubcoreMesh(**_MESH),
        scratch_types=(
            [pltpu.VMEM((2, _CH), jnp.int32)] * _IB
            + [pltpu.VMEM((_CH, D), jnp.float32)] * _NBUF
            + [pltpu.VMEM_SHARED((_NA, D), jnp.float32)]
            + [pltpu.SemaphoreType.DMA]
            + sems
        ),
    )(hp, edges4)


def _tc_first_body(x_ref, w_ref, degp_ref, hp_ref, dinv_ref):
    deg = jnp.sum(degp_ref[...], axis=1, keepdims=True) + 1.0
    dinv = lax.rsqrt(deg)
    h = jnp.dot(x_ref[...], w_ref[...], preferred_element_type=jnp.float32)
    hp_ref[...] = h * dinv
    dinv_ref[...] = dinv


def _tc_first(x, w, degp_t):
    return pl.pallas_call(
        _tc_first_body,
        grid=(N // BN,),
        in_specs=[
            pl.BlockSpec((BN, D), lambda i: (i, 0)),
            pl.BlockSpec((D, D), lambda i: (0, 0)),
            pl.BlockSpec((BN, NW), lambda i: (i, 0)),
        ],
        out_specs=[
            pl.BlockSpec((BN, D), lambda i: (i, 0)),
            pl.BlockSpec((BN, 1), lambda i: (i, 0)),
        ],
        out_shape=[
            jax.ShapeDtypeStruct((N, D), jnp.float32),
            jax.ShapeDtypeStruct((N, 1), jnp.float32),
        ],
    )(x, w, degp_t)


def _tc_mid_body(s_ref, hp_ref, dinv_ref, b_ref, w_ref, out_ref):
    t = s_ref[0] + s_ref[1] + hp_ref[...]
    x = jnp.maximum(t * dinv_ref[...] + b_ref[...], 0.0)
    out_ref[...] = (
        jnp.dot(x, w_ref[...], preferred_element_type=jnp.float32) * dinv_ref[...]
    )


def _tc_mid(s_parts, hp, dinv, b, w):
    return pl.pallas_call(
        _tc_mid_body,
        grid=(N // BN,),
        in_specs=[
            pl.BlockSpec((NC, BN, D), lambda i: (0, i, 0)),
            pl.BlockSpec((BN, D), lambda i: (i, 0)),
            pl.BlockSpec((BN, 1), lambda i: (i, 0)),
            pl.BlockSpec((1, D), lambda i: (0, 0)),
            pl.BlockSpec((D, D), lambda i: (0, 0)),
        ],
        out_specs=pl.BlockSpec((BN, D), lambda i: (i, 0)),
        out_shape=jax.ShapeDtypeStruct((N, D), jnp.float32),
    )(s_parts, hp, dinv, b, w)


def _tc_head_body(s_ref, hp_ref, dinv_ref, b3_ref, wf1_ref, bf1_ref, batch_ref,
                  wf2_ref, bf2_ref, out_ref, pool_ref, cnt_ref):
    i = pl.program_id(0)
    t = s_ref[0] + s_ref[1] + hp_ref[...]
    x3 = jnp.maximum(t * dinv_ref[...] + b3_ref[...], 0.0)
    h = jnp.dot(x3, wf1_ref[...], preferred_element_type=jnp.float32)
    h = jnp.maximum(h + bf1_ref[...], 0.0)
    onehot = (batch_ref[...] == lax.broadcasted_iota(jnp.int32, (BN, G), 1))
    onehot = onehot.astype(jnp.float32)
    psum = lax.dot_general(onehot, h, (((0,), (0,)), ((), ())),
                           preferred_element_type=jnp.float32)

    @pl.when(i == 0)
    def _():
        pool_ref[...] = jnp.zeros_like(pool_ref)
        cnt_ref[...] = jnp.zeros_like(cnt_ref)

    pool_ref[...] += psum
    cnt_ref[...] += jnp.sum(onehot, axis=0, keepdims=True)

    @pl.when(i == N // BN - 1)
    def _():
        cnt = jnp.transpose(cnt_ref[...])
        mean = pool_ref[...] / jnp.maximum(cnt, 1.0)
        logits = jnp.dot(mean, wf2_ref[...],
                         preferred_element_type=jnp.float32) + bf2_ref[...]
        m = jnp.max(logits, axis=1, keepdims=True)
        ex = jnp.exp(logits - m)
        out_ref[...] = ex / jnp.sum(ex, axis=1, keepdims=True)


def _tc_head(s_parts, hp, dinv, b3, wf1, bf1, batch2d, wf2, bf2):
    return pl.pallas_call(
        _tc_head_body,
        grid=(N // BN,),
        in_specs=[
            pl.BlockSpec((NC, BN, D), lambda i: (0, i, 0)),
            pl.BlockSpec((BN, D), lambda i: (i, 0)),
            pl.BlockSpec((BN, 1), lambda i: (i, 0)),
            pl.BlockSpec((1, D), lambda i: (0, 0)),
            pl.BlockSpec((D, D), lambda i: (0, 0)),
            pl.BlockSpec((1, D), lambda i: (0, 0)),
            pl.BlockSpec((BN, 1), lambda i: (i, 0)),
            pl.BlockSpec((D, 2), lambda i: (0, 0)),
            pl.BlockSpec((1, 2), lambda i: (0, 0)),
        ],
        out_specs=pl.BlockSpec((G, 2), lambda i: (0, 0)),
        out_shape=jax.ShapeDtypeStruct((G, 2), jnp.float32),
        scratch_shapes=[
            pltpu.VMEM((G, D), jnp.float32),
            pltpu.VMEM((1, G), jnp.float32),
        ],
    )(s_parts, hp, dinv, b3, wf1, bf1, batch2d, wf2, bf2)


def kernel(x, edge_index, batch, W1, b1, W2, b2, W3, b3, Wf1, bf1, Wf2, bf2):
    row = edge_index[0]
    col = edge_index[1]
    pad = _NCH * _CH - _EPT
    rowp = jnp.pad(row.reshape(NW, _EPT), ((0, 0), (0, pad)))
    colp = jnp.pad(col.reshape(NW, _EPT), ((0, 0), (0, pad)),
                   constant_values=N)
    edges4 = jnp.stack([rowp.reshape(NW, _NCH, _CH),
                        colp.reshape(NW, _NCH, _CH)], axis=2)

    degp = _sc_degree(col)
    degp_t = degp.T

    hp1, dinv = _tc_first(x, W1, degp_t)
    s1 = _sc_scatter(hp1, edges4)
    hp2 = _tc_mid(s1, hp1, dinv, b1.reshape(1, D), W2)
    s2 = _sc_scatter(hp2, edges4)
    hp3 = _tc_mid(s2, hp2, dinv, b2.reshape(1, D), W3)
    s3 = _sc_scatter(hp3, edges4)
    return _tc_head(s3, hp3, dinv, b3.reshape(1, D), Wf1,
                    bf1.reshape(1, D), batch.reshape(N, 1),
                    Wf2, bf2.reshape(1, 2))
